# baseline (device time: 224942 ns/iter reference)
import jax
import jax.numpy as jnp
from jax import lax
from jax.experimental import pallas as pl
from jax.experimental.pallas import tpu as pltpu

N_DEV = 8
SQ = 256
SKV = 4096
NSLAB = 1408
KP = 3 * NSLAB
HQ = 8
DH = 128
D = HQ * DH
SCALE = 0.08838834764831843
NEG = -1e9
M_INIT = -1e30

_GROUPS = ((0, 128, 0), (128, 64, 1), (192, 64, 2))
_PERM_I = (0, 3, 1, 2)


def _body(x_ref, wq_ref, k_ref, v_ref, wo_ref, out_ref,
          q_buf, acc_buf, st_buf, pv_buf, loc_ref,
          sq_sem, rq_sem, sa_sem, ra_sem, ssem, rsem):
    d = lax.axis_index("i")
    left = lax.rem(d + N_DEV - 1, N_DEV)
    right = lax.rem(d + 1, N_DEV)

    def q_rdma(slot):
        return pltpu.make_async_remote_copy(
            src_ref=q_buf.at[slot], dst_ref=q_buf.at[slot + 1],
            send_sem=sq_sem.at[slot], recv_sem=rq_sem.at[slot + 1],
            device_id=(right,), device_id_type=pl.DeviceIdType.MESH)

    def acc_rdma(slot):
        return pltpu.make_async_remote_copy(
            src_ref=acc_buf.at[slot], dst_ref=acc_buf.at[slot + 1],
            send_sem=sa_sem.at[slot], recv_sem=ra_sem.at[slot + 1],
            device_id=(right,), device_id_type=pl.DeviceIdType.MESH)

    def st_rdma(slot):
        return pltpu.make_async_remote_copy(
            src_ref=st_buf.at[slot], dst_ref=st_buf.at[slot + 1],
            send_sem=ssem.at[slot], recv_sem=rsem.at[slot + 1],
            device_id=(right,), device_id_type=pl.DeviceIdType.MESH)

    barrier_sem = pltpu.get_barrier_semaphore()
    for nbr in (left, right):
        pl.semaphore_signal(barrier_sem, inc=1, device_id=(nbr,),
                            device_id_type=pl.DeviceIdType.MESH)
    pl.semaphore_wait(barrier_sem, 2)

    q = jnp.dot(x_ref[...], wq_ref[...], preferred_element_type=jnp.float32)
    q_buf[0] = q.astype(jnp.bfloat16)
    acc_buf[0] = jnp.zeros((SQ, D), jnp.bfloat16)
    st_buf[0, 0] = jnp.full((SQ, HQ), M_INIT, jnp.float32)
    st_buf[0, 1] = jnp.zeros((SQ, HQ), jnp.float32)

    def hop_body(hop, carry):
        @pl.when(hop > 0)
        def _():
            q_rdma(hop - 1).wait_recv()

        @pl.when(hop < N_DEV - 1)
        def _():
            q_rdma(hop).start()

        owner = lax.rem(d - hop + N_DEV, N_DEV)
        o4 = owner * 4

        for r0, rn, delta in _GROUPS:
            cg = lax.rem(3 - lax.rem(o4 + delta, 3), 3)
            start = lax.rem(cg - d + 24, 3)
            vk = jnp.where(start == 0, NSLAB, NSLAB - 64)
            col = lax.broadcasted_iota(jnp.int32, (rn, NSLAB), 1)
            for h in range(HQ):
                qh = q_buf[hop, r0:r0 + rn, h * DH:(h + 1) * DH]
                ks = k_ref[h, pl.ds(cg * NSLAB, NSLAB), :]
                s = lax.dot_general(
                    qh, ks, (((1,), (1,)), ((), ())),
                    preferred_element_type=jnp.float32) * SCALE
                s = jnp.where(col < vk, s, NEG)
                m_loc = jnp.max(s, axis=1, keepdims=True)
                p = jnp.exp(s - m_loc)
                l_loc = jnp.sum(p, axis=1, keepdims=True)
                vs = v_ref[h, pl.ds(cg * NSLAB, NSLAB), :]
                pv = lax.dot_general(
                    p.astype(jnp.bfloat16), vs, (((1,), (0,)), ((), ())),
                    preferred_element_type=jnp.float32)
                pv_buf[r0:r0 + rn, h * DH:(h + 1) * DH] = pv
                loc_ref[0, r0:r0 + rn, h:h + 1] = m_loc
                loc_ref[1, r0:r0 + rn, h:h + 1] = l_loc

        @pl.when(d == 0)
        def _():
            row = lax.broadcasted_iota(jnp.int32, (SQ, 320), 0)
            colx = lax.broadcasted_iota(jnp.int32, (SQ, 320), 1)
            rowblk = row // 64
            sel = jnp.where(rowblk == 0, 0,
                            jnp.where(rowblk == 1, 3,
                                      jnp.where(rowblk == 2, 1, 2)))
            qbr = o4 + sel
            cond_row = lax.rem(qbr, 3) != 0
            keep_x = cond_row & ((colx >= 256) | (colx // 64 == rowblk))

            offs = []
            for p_slot in range(4):
                qb = o4 + _PERM_I[p_slot]
                offs.append(lax.rem(qb, 3) * NSLAB + (qb // 3) * 64)
            for h in range(HQ):
                kx = jnp.concatenate(
                    [k_ref[h, pl.ds(off, 64), :] for off in offs]
                    + [k_ref[h, pl.ds(0, 64), :]], axis=0)
                vx = jnp.concatenate(
                    [v_ref[h, pl.ds(off, 64), :] for off in offs]
                    + [v_ref[h, pl.ds(0, 64), :]], axis=0)
                qh = q_buf[hop, :, h * DH:(h + 1) * DH]
                s2 = lax.dot_general(
                    qh, kx, (((1,), (1,)), ((), ())),
                    preferred_element_type=jnp.float32) * SCALE
                s2 = jnp.where(keep_x, s2, NEG)
                m2 = jnp.max(s2, axis=1, keepdims=True)
                p2 = jnp.exp(s2 - m2)
                l2 = jnp.sum(p2, axis=1, keepdims=True)
                pv2 = lax.dot_general(
                    p2.astype(jnp.bfloat16), vx, (((1,), (0,)), ((), ())),
                    preferred_element_type=jnp.float32)
                m1 = loc_ref[0, :, h:h + 1]
                l1 = loc_ref[1, :, h:h + 1]
                m_new = jnp.maximum(m1, m2)
                a1 = jnp.exp(m1 - m_new)
                a2 = jnp.exp(m2 - m_new)
                pv1 = pv_buf[:, h * DH:(h + 1) * DH]
                pv_buf[:, h * DH:(h + 1) * DH] = pv1 * a1 + pv2 * a2
                loc_ref[0, :, h:h + 1] = m_new
                loc_ref[1, :, h:h + 1] = l1 * a1 + l2 * a2

        @pl.when(hop > 0)
        def _():
            acc_rdma(hop - 1).wait_send()
            st_rdma(hop - 1).wait_send()

            @pl.when(hop - 1 < N_DEV - 1)
            def _():
                q_rdma(hop - 1).wait_send()

        @pl.when(hop > 0)
        def _():
            acc_rdma(hop - 1).wait_recv()
            st_rdma(hop - 1).wait_recv()

        for h in range(HQ):
            m_prev = st_buf[hop, 0, :, h:h + 1]
            l_prev = st_buf[hop, 1, :, h:h + 1]
            m_loc = loc_ref[0, :, h:h + 1]
            l_loc = loc_ref[1, :, h:h + 1]
            m_new = jnp.maximum(m_prev, m_loc)
            alpha = jnp.exp(m_prev - m_new)
            beta = jnp.exp(m_loc - m_new)
            acc = acc_buf[hop, :, h * DH:(h + 1) * DH].astype(jnp.float32)
            pv = pv_buf[:, h * DH:(h + 1) * DH]
            acc_buf[hop, :, h * DH:(h + 1) * DH] = (
                acc * alpha + pv * beta).astype(jnp.bfloat16)
            st_buf[hop, 0, :, h:h + 1] = m_new
            st_buf[hop, 1, :, h:h + 1] = l_prev * alpha + l_loc * beta

        acc_rdma(hop).start()
        st_rdma(hop).start()
        return carry

    lax.fori_loop(0, N_DEV, hop_body, 0)

    acc_rdma(N_DEV - 1).wait_send()
    st_rdma(N_DEV - 1).wait_send()
    acc_rdma(N_DEV - 1).wait_recv()
    st_rdma(N_DEV - 1).wait_recv()

    parts = []
    for h in range(HQ):
        acc = acc_buf[N_DEV, :, h * DH:(h + 1) * DH].astype(jnp.float32)
        l = st_buf[N_DEV, 1, :, h:h + 1]
        parts.append(acc / l)
    ctx = jnp.concatenate(parts, axis=1)
    out_ref[...] = jnp.dot(ctx.astype(jnp.bfloat16), wo_ref[...],
                           preferred_element_type=jnp.float32)


def kernel(x, Wq, K_ext, V_ext, Wo):
    d = lax.axis_index("i")

    parts = []
    for c in range(3):
        start = lax.rem(c - d + 24, 3)
        t = jnp.arange(22, dtype=jnp.int32)
        jb = start + 3 * t
        jb = jnp.where(jb < 64, jb, 0)
        parts.append((jb[:, None] * 64
                      + jnp.arange(64, dtype=jnp.int32)[None, :]).reshape(-1))
    keyidx = jnp.concatenate(parts)

    kp = jnp.take(K_ext[0], keyidx, axis=0).astype(jnp.bfloat16)
    vp = jnp.take(V_ext[0], keyidx, axis=0).astype(jnp.bfloat16)
    kp = jnp.transpose(kp, (1, 0, 2))
    vp = jnp.transpose(vp, (1, 0, 2))

    x2 = x[0]
    xp = jnp.concatenate([x2[0:64], x2[192:256], x2[64:128], x2[128:192]])
    xb = xp.astype(jnp.bfloat16)
    wq = Wq.astype(jnp.bfloat16)
    wo = Wo.astype(jnp.bfloat16)

    out = pl.pallas_call(
        _body,
        out_shape=jax.ShapeDtypeStruct((SQ, D), jnp.float32),
        in_specs=[pl.BlockSpec(memory_space=pltpu.VMEM)] * 5,
        out_specs=pl.BlockSpec(memory_space=pltpu.VMEM),
        scratch_shapes=[
            pltpu.VMEM((N_DEV + 1, SQ, D), jnp.bfloat16),
            pltpu.VMEM((N_DEV + 1, SQ, D), jnp.bfloat16),
            pltpu.VMEM((N_DEV + 1, 2, SQ, HQ), jnp.float32),
            pltpu.VMEM((SQ, D), jnp.float32),
            pltpu.VMEM((2, SQ, HQ), jnp.float32),
            pltpu.SemaphoreType.DMA((N_DEV + 1,)),
            pltpu.SemaphoreType.DMA((N_DEV + 1,)),
            pltpu.SemaphoreType.DMA((N_DEV + 1,)),
            pltpu.SemaphoreType.DMA((N_DEV + 1,)),
            pltpu.SemaphoreType.DMA((N_DEV + 1,)),
            pltpu.SemaphoreType.DMA((N_DEV + 1,)),
        ],
        compiler_params=pltpu.CompilerParams(
            collective_id=0,
            vmem_limit_bytes=100 * 1024 * 1024,
        ),
    )(xb, wq, kp, vp, wo)

    out = jnp.concatenate([out[0:64], out[128:192], out[192:256], out[64:128]])
    return out[None]


# device time: 188699 ns/iter; 1.1921x vs baseline; 1.1921x over previous
import jax
import jax.numpy as jnp
from jax import lax
from jax.experimental import pallas as pl
from jax.experimental.pallas import tpu as pltpu

N_DEV = 8
SQ = 256
SKV = 4096
HQ = 8
DH = 128
D = HQ * DH
SCALE = 0.08838834764831843
NEG = -1e9
M_INIT = -1e30


def _body(x_ref, wq_ref, k_ref, v_ref, wo_ref, out_ref,
          q_buf, acc_buf, st_buf, bias_ref, pv_buf, loc_ref,
          sq_sem, rq_sem, sa_sem, ra_sem, ssem, rsem):
    d = lax.axis_index("i")
    left = lax.rem(d + N_DEV - 1, N_DEV)
    right = lax.rem(d + 1, N_DEV)

    def q_rdma(slot):
        return pltpu.make_async_remote_copy(
            src_ref=q_buf.at[slot], dst_ref=q_buf.at[slot + 1],
            send_sem=sq_sem.at[slot], recv_sem=rq_sem.at[slot + 1],
            device_id=(right,), device_id_type=pl.DeviceIdType.MESH)

    def acc_rdma(slot):
        return pltpu.make_async_remote_copy(
            src_ref=acc_buf.at[slot], dst_ref=acc_buf.at[slot + 1],
            send_sem=sa_sem.at[slot], recv_sem=ra_sem.at[slot + 1],
            device_id=(right,), device_id_type=pl.DeviceIdType.MESH)

    def st_rdma(slot):
        return pltpu.make_async_remote_copy(
            src_ref=st_buf.at[slot], dst_ref=st_buf.at[slot + 1],
            send_sem=ssem.at[slot], recv_sem=rsem.at[slot + 1],
            device_id=(right,), device_id_type=pl.DeviceIdType.MESH)

    barrier_sem = pltpu.get_barrier_semaphore()
    for nbr in (left, right):
        pl.semaphore_signal(barrier_sem, inc=1, device_id=(nbr,),
                            device_id_type=pl.DeviceIdType.MESH)
    pl.semaphore_wait(barrier_sem, 2)

    q = jnp.dot(x_ref[...], wq_ref[...], preferred_element_type=jnp.float32)
    q_buf[0] = (q * SCALE).astype(jnp.bfloat16)
    acc_buf[0] = jnp.zeros((SQ, D), jnp.bfloat16)
    st_buf[0, 0] = jnp.full((SQ, HQ), M_INIT, jnp.float32)
    st_buf[0, 1] = jnp.zeros((SQ, HQ), jnp.float32)

    def hop_body(hop, carry):
        @pl.when(hop > 0)
        def _():
            q_rdma(hop - 1).wait_recv()

        @pl.when(hop < N_DEV - 1)
        def _():
            q_rdma(hop).start()

        owner = lax.rem(d - hop + N_DEV, N_DEV)
        ri = lax.broadcasted_iota(jnp.int32, (SQ, SKV), 0)
        ci = lax.broadcasted_iota(jnp.int32, (SQ, SKV), 1)
        qb = owner * (SQ // 64) + ri // 64
        kb = d * (SKV // 64) + ci // 64
        keep = (qb == kb) | (kb == 0) | (lax.rem(qb + kb, 3) == 0)
        bias_ref[...] = jnp.where(keep, 0.0, NEG).astype(jnp.float32)

        for h in range(HQ):
            qh = q_buf[hop, :, h * DH:(h + 1) * DH]
            s = lax.dot_general(
                qh, k_ref[h], (((1,), (1,)), ((), ())),
                preferred_element_type=jnp.float32)
            s = s + bias_ref[...]
            m_loc = jnp.max(s, axis=1, keepdims=True)
            p = jnp.exp(s - m_loc)
            l_loc = jnp.sum(p, axis=1, keepdims=True)
            pv = lax.dot_general(
                p.astype(jnp.bfloat16), v_ref[h], (((1,), (0,)), ((), ())),
                preferred_element_type=jnp.float32)
            pv_buf[:, h * DH:(h + 1) * DH] = pv
            loc_ref[0, :, h:h + 1] = m_loc
            loc_ref[1, :, h:h + 1] = l_loc

        @pl.when(hop > 0)
        def _():
            acc_rdma(hop - 1).wait_send()
            st_rdma(hop - 1).wait_send()

            @pl.when(hop - 1 < N_DEV - 1)
            def _():
                q_rdma(hop - 1).wait_send()

        @pl.when(hop > 0)
        def _():
            acc_rdma(hop - 1).wait_recv()
            st_rdma(hop - 1).wait_recv()

        for h in range(HQ):
            m_prev = st_buf[hop, 0, :, h:h + 1]
            l_prev = st_buf[hop, 1, :, h:h + 1]
            m_loc = loc_ref[0, :, h:h + 1]
            l_loc = loc_ref[1, :, h:h + 1]
            m_new = jnp.maximum(m_prev, m_loc)
            alpha = jnp.exp(m_prev - m_new)
            beta = jnp.exp(m_loc - m_new)
            acc = acc_buf[hop, :, h * DH:(h + 1) * DH].astype(jnp.float32)
            pv = pv_buf[:, h * DH:(h + 1) * DH]
            acc_buf[hop, :, h * DH:(h + 1) * DH] = (
                acc * alpha + pv * beta).astype(jnp.bfloat16)
            st_buf[hop, 0, :, h:h + 1] = m_new
            st_buf[hop, 1, :, h:h + 1] = l_prev * alpha + l_loc * beta

        acc_rdma(hop).start()
        st_rdma(hop).start()
        return carry

    lax.fori_loop(0, N_DEV, hop_body, 0)

    acc_rdma(N_DEV - 1).wait_send()
    st_rdma(N_DEV - 1).wait_send()
    acc_rdma(N_DEV - 1).wait_recv()
    st_rdma(N_DEV - 1).wait_recv()

    parts = []
    for h in range(HQ):
        acc = acc_buf[N_DEV, :, h * DH:(h + 1) * DH].astype(jnp.float32)
        l = st_buf[N_DEV, 1, :, h:h + 1]
        parts.append(acc / l)
    ctx = jnp.concatenate(parts, axis=1)
    out_ref[...] = jnp.dot(ctx.astype(jnp.bfloat16), wo_ref[...],
                           preferred_element_type=jnp.float32)


def kernel(x, Wq, K_ext, V_ext, Wo):
    xb = x[0].astype(jnp.bfloat16)
    wq = Wq.astype(jnp.bfloat16)
    wo = Wo.astype(jnp.bfloat16)
    kb = jnp.transpose(K_ext[0].astype(jnp.bfloat16), (1, 0, 2))
    vb = jnp.transpose(V_ext[0].astype(jnp.bfloat16), (1, 0, 2))

    out = pl.pallas_call(
        _body,
        out_shape=jax.ShapeDtypeStruct((SQ, D), jnp.float32),
        in_specs=[pl.BlockSpec(memory_space=pltpu.VMEM)] * 5,
        out_specs=pl.BlockSpec(memory_space=pltpu.VMEM),
        scratch_shapes=[
            pltpu.VMEM((N_DEV + 1, SQ, D), jnp.bfloat16),
            pltpu.VMEM((N_DEV + 1, SQ, D), jnp.bfloat16),
            pltpu.VMEM((N_DEV + 1, 2, SQ, HQ), jnp.float32),
            pltpu.VMEM((SQ, SKV), jnp.float32),
            pltpu.VMEM((SQ, D), jnp.float32),
            pltpu.VMEM((2, SQ, HQ), jnp.float32),
            pltpu.SemaphoreType.DMA((N_DEV + 1,)),
            pltpu.SemaphoreType.DMA((N_DEV + 1,)),
            pltpu.SemaphoreType.DMA((N_DEV + 1,)),
            pltpu.SemaphoreType.DMA((N_DEV + 1,)),
            pltpu.SemaphoreType.DMA((N_DEV + 1,)),
            pltpu.SemaphoreType.DMA((N_DEV + 1,)),
        ],
        compiler_params=pltpu.CompilerParams(
            collective_id=0,
            vmem_limit_bytes=100 * 1024 * 1024,
        ),
    )(xb, wq, kb, vb, wo)
    return out[None]
